# baseline (speedup 1.0000x reference)
import functools

import jax
import jax.numpy as jnp
from jax import lax
from jax.experimental import pallas as pl
from jax.experimental.pallas import tpu as pltpu
from jax.experimental.pallas import tpu_sc as plsc

B = 16384
D = 256
LANES = 16
NC = 2
NS = 16
NW = NC * NS
BPW = B // NW
C = 32
NCH = BPW // C
NSLOT = 6
LA = 4
KCH = D // LANES


def _dot_chunk(ubuf, ibuf, xui_v, slot, c):
    lane = lax.iota(jnp.int32, LANES)

    def group_body(g, _):
        def row_body(j, vals):
            r = g * LANES + j
            acc = ubuf[slot, r, pl.ds(0, LANES)] * ibuf[slot, r, pl.ds(0, LANES)]
            for k in range(1, KCH):
                acc = acc + (ubuf[slot, r, pl.ds(k * LANES, LANES)]
                             * ibuf[slot, r, pl.ds(k * LANES, LANES)])
            return jnp.where(lane == j, jnp.sum(acc), vals)

        vals = lax.fori_loop(0, LANES, row_body,
                             jnp.zeros((LANES,), jnp.float32), unroll=4)
        xui_v[pl.ds(c * C + g * LANES, LANES)] = vals
        return 0

    lax.fori_loop(0, C // LANES, group_body, 0)


@functools.partial(
    pl.kernel,
    out_type=(
        jax.ShapeDtypeStruct((B,), jnp.float32),
        jax.ShapeDtypeStruct((B, D), jnp.float32),
        jax.ShapeDtypeStruct((B, D), jnp.float32),
    ),
    mesh=plsc.VectorSubcoreMesh(core_axis_name="c", subcore_axis_name="s"),
    compiler_params=pltpu.CompilerParams(needs_layout_passes=False),
    scratch_types=[
        pltpu.VMEM((BPW,), jnp.int32),
        pltpu.VMEM((BPW,), jnp.int32),
        pltpu.VMEM((NSLOT, C, D), jnp.float32),
        pltpu.VMEM((NSLOT, C, D), jnp.float32),
        pltpu.VMEM((BPW,), jnp.float32),
        [pltpu.SemaphoreType.DMA] * NSLOT,
        [pltpu.SemaphoreType.DMA] * NSLOT,
    ],
)
def _ngcf_sc(user_h, item_h, Gu_h, Gi_h, xui_o, gu_o, gi_o,
             uidx, iidx, ubuf, ibuf, xui_v, gsems, wsems):
    wid = lax.axis_index("s") * NC + lax.axis_index("c")
    base = wid * BPW

    pltpu.sync_copy(user_h.at[pl.ds(base, BPW)], uidx)
    pltpu.sync_copy(item_h.at[pl.ds(base, BPW)], iidx)

    def start_gather(c):
        slot = c % NSLOT
        cu = pltpu.async_copy(Gu_h.at[uidx.at[pl.ds(c * C, C)]],
                              ubuf.at[slot], gsems[slot])
        ci = pltpu.async_copy(Gi_h.at[iidx.at[pl.ds(c * C, C)]],
                              ibuf.at[slot], gsems[slot])
        return cu, ci

    def start_write(c):
        slot = c % NSLOT
        wu = pltpu.async_copy(ubuf.at[slot], gu_o.at[pl.ds(base + c * C, C)],
                              wsems[slot])
        wi = pltpu.async_copy(ibuf.at[slot], gi_o.at[pl.ds(base + c * C, C)],
                              wsems[slot])
        return wu, wi

    gpend = {c: start_gather(c) for c in range(LA)}
    wpend = {}
    for c in range(NCH):
        cu, ci = gpend.pop(c)
        cu.wait()
        ci.wait()
        wpend[c] = start_write(c)
        if c + LA < NCH:
            prev = c + LA - NSLOT
            if prev in wpend:
                wu, wi = wpend.pop(prev)
                wu.wait()
                wi.wait()
            gpend[c + LA] = start_gather(c + LA)
        _dot_chunk(ubuf, ibuf, xui_v, c % NSLOT, c)

    for wu, wi in wpend.values():
        wu.wait()
        wi.wait()
    pltpu.sync_copy(xui_v, xui_o.at[pl.ds(base, BPW)])


def kernel(user, item, Gu, Gi):
    xui, gamma_u, gamma_i = _ngcf_sc(user, item, Gu, Gi)
    return (xui, gamma_u, gamma_i)


# speedup vs baseline: 1.0550x; 1.0550x over previous
import functools

import jax
import jax.numpy as jnp
from jax import lax
from jax.experimental import pallas as pl
from jax.experimental.pallas import tpu as pltpu
from jax.experimental.pallas import tpu_sc as plsc

B = 16384
D = 256
LANES = 16
NC = 2
NS = 16
NW = NC * NS
BPW = B // NW
C = 64
NCH = BPW // C
NSLOT = 3
LA = 2
KCH = D // LANES


def _dot_chunk(ubuf, ibuf, xui_v, slot, c):
    lane = lax.iota(jnp.int32, LANES)

    def group_body(g, _):
        def row_body(j, vals):
            r = g * LANES + j
            acc = ubuf[slot, r, pl.ds(0, LANES)] * ibuf[slot, r, pl.ds(0, LANES)]
            for k in range(1, KCH):
                acc = acc + (ubuf[slot, r, pl.ds(k * LANES, LANES)]
                             * ibuf[slot, r, pl.ds(k * LANES, LANES)])
            return jnp.where(lane == j, jnp.sum(acc), vals)

        vals = lax.fori_loop(0, LANES, row_body,
                             jnp.zeros((LANES,), jnp.float32), unroll=8)
        xui_v[pl.ds(c * C + g * LANES, LANES)] = vals
        return 0

    lax.fori_loop(0, C // LANES, group_body, 0)


@functools.partial(
    pl.kernel,
    out_type=(
        jax.ShapeDtypeStruct((B,), jnp.float32),
        jax.ShapeDtypeStruct((B, D), jnp.float32),
        jax.ShapeDtypeStruct((B, D), jnp.float32),
    ),
    mesh=plsc.VectorSubcoreMesh(core_axis_name="c", subcore_axis_name="s"),
    compiler_params=pltpu.CompilerParams(needs_layout_passes=False),
    scratch_types=[
        pltpu.VMEM((BPW,), jnp.int32),
        pltpu.VMEM((BPW,), jnp.int32),
        pltpu.VMEM((NSLOT, C, D), jnp.float32),
        pltpu.VMEM((NSLOT, C, D), jnp.float32),
        pltpu.VMEM((BPW,), jnp.float32),
        [pltpu.SemaphoreType.DMA] * NSLOT,
        [pltpu.SemaphoreType.DMA] * NSLOT,
    ],
)
def _ngcf_sc(user_h, item_h, Gu_h, Gi_h, xui_o, gu_o, gi_o,
             uidx, iidx, ubuf, ibuf, xui_v, gsems, wsems):
    wid = lax.axis_index("s") * NC + lax.axis_index("c")
    base = wid * BPW

    pltpu.sync_copy(user_h.at[pl.ds(base, BPW)], uidx)
    pltpu.sync_copy(item_h.at[pl.ds(base, BPW)], iidx)

    def start_gather(c):
        slot = c % NSLOT
        cu = pltpu.async_copy(Gu_h.at[uidx.at[pl.ds(c * C, C)]],
                              ubuf.at[slot], gsems[slot])
        ci = pltpu.async_copy(Gi_h.at[iidx.at[pl.ds(c * C, C)]],
                              ibuf.at[slot], gsems[slot])
        return cu, ci

    def start_write(c):
        slot = c % NSLOT
        wu = pltpu.async_copy(ubuf.at[slot], gu_o.at[pl.ds(base + c * C, C)],
                              wsems[slot])
        wi = pltpu.async_copy(ibuf.at[slot], gi_o.at[pl.ds(base + c * C, C)],
                              wsems[slot])
        return wu, wi

    gpend = {c: start_gather(c) for c in range(LA)}
    wpend = {}
    for c in range(NCH):
        cu, ci = gpend.pop(c)
        cu.wait()
        ci.wait()
        wpend[c] = start_write(c)
        if c + LA < NCH:
            prev = c + LA - NSLOT
            if prev in wpend:
                wu, wi = wpend.pop(prev)
                wu.wait()
                wi.wait()
            gpend[c + LA] = start_gather(c + LA)
        _dot_chunk(ubuf, ibuf, xui_v, c % NSLOT, c)

    for wu, wi in wpend.values():
        wu.wait()
        wi.wait()
    pltpu.sync_copy(xui_v, xui_o.at[pl.ds(base, BPW)])


def kernel(user, item, Gu, Gi):
    xui, gamma_u, gamma_i = _ngcf_sc(user, item, Gu, Gi)
    return (xui, gamma_u, gamma_i)


# speedup vs baseline: 1.0632x; 1.0077x over previous
import functools

import jax
import jax.numpy as jnp
from jax import lax
from jax.experimental import pallas as pl
from jax.experimental.pallas import tpu as pltpu
from jax.experimental.pallas import tpu_sc as plsc

B = 16384
D = 256
LANES = 16
NC = 2
NS = 16
NW = NC * NS
BPW = B // NW
C = 64
NCH = BPW // C
NSLOT = 3
LA = 2
KCH = D // LANES


def _dot_chunk(ubuf, ibuf, xui_v, slot, c):
    last = lax.iota(jnp.int32, LANES) == (LANES - 1)

    def row_body(r):
        acc = ubuf[slot, r, pl.ds(0, LANES)] * ibuf[slot, r, pl.ds(0, LANES)]
        for k in range(1, KCH):
            acc = acc + (ubuf[slot, r, pl.ds(k * LANES, LANES)]
                         * ibuf[slot, r, pl.ds(k * LANES, LANES)])
        s = plsc.cumsum(acc)
        plsc.store_scatter(xui_v, [jnp.full((LANES,), c * C, jnp.int32) + r],
                           s, mask=last)

    plsc.parallel_loop(0, C, unroll=4)(row_body)


@functools.partial(
    pl.kernel,
    out_type=(
        jax.ShapeDtypeStruct((B,), jnp.float32),
        jax.ShapeDtypeStruct((B, D), jnp.float32),
        jax.ShapeDtypeStruct((B, D), jnp.float32),
    ),
    mesh=plsc.VectorSubcoreMesh(core_axis_name="c", subcore_axis_name="s"),
    compiler_params=pltpu.CompilerParams(needs_layout_passes=False),
    scratch_types=[
        pltpu.VMEM((BPW,), jnp.int32),
        pltpu.VMEM((BPW,), jnp.int32),
        pltpu.VMEM((NSLOT, C, D), jnp.float32),
        pltpu.VMEM((NSLOT, C, D), jnp.float32),
        pltpu.VMEM((BPW,), jnp.float32),
        [pltpu.SemaphoreType.DMA] * NSLOT,
        [pltpu.SemaphoreType.DMA] * NSLOT,
    ],
)
def _ngcf_sc(user_h, item_h, Gu_h, Gi_h, xui_o, gu_o, gi_o,
             uidx, iidx, ubuf, ibuf, xui_v, gsems, wsems):
    wid = lax.axis_index("s") * NC + lax.axis_index("c")
    base = wid * BPW

    pltpu.sync_copy(user_h.at[pl.ds(base, BPW)], uidx)
    pltpu.sync_copy(item_h.at[pl.ds(base, BPW)], iidx)

    def start_gather(c):
        slot = c % NSLOT
        cu = pltpu.async_copy(Gu_h.at[uidx.at[pl.ds(c * C, C)]],
                              ubuf.at[slot], gsems[slot])
        ci = pltpu.async_copy(Gi_h.at[iidx.at[pl.ds(c * C, C)]],
                              ibuf.at[slot], gsems[slot])
        return cu, ci

    def start_write(c):
        slot = c % NSLOT
        wu = pltpu.async_copy(ubuf.at[slot], gu_o.at[pl.ds(base + c * C, C)],
                              wsems[slot])
        wi = pltpu.async_copy(ibuf.at[slot], gi_o.at[pl.ds(base + c * C, C)],
                              wsems[slot])
        return wu, wi

    gpend = {c: start_gather(c) for c in range(LA)}
    wpend = {}
    for c in range(NCH):
        cu, ci = gpend.pop(c)
        cu.wait()
        ci.wait()
        wpend[c] = start_write(c)
        if c + LA < NCH:
            prev = c + LA - NSLOT
            if prev in wpend:
                wu, wi = wpend.pop(prev)
                wu.wait()
                wi.wait()
            gpend[c + LA] = start_gather(c + LA)
        _dot_chunk(ubuf, ibuf, xui_v, c % NSLOT, c)

    for wu, wi in wpend.values():
        wu.wait()
        wi.wait()
    pltpu.sync_copy(xui_v, xui_o.at[pl.ds(base, BPW)])


def kernel(user, item, Gu, Gi):
    xui, gamma_u, gamma_i = _ngcf_sc(user, item, Gu, Gi)
    return (xui, gamma_u, gamma_i)


# speedup vs baseline: 1.0732x; 1.0095x over previous
import functools

import jax
import jax.numpy as jnp
from jax import lax
from jax.experimental import pallas as pl
from jax.experimental.pallas import tpu as pltpu
from jax.experimental.pallas import tpu_sc as plsc

B = 16384
D = 256
LANES = 16
NC = 2
NS = 16
NW = NC * NS
BPW = B // NW
C = 64
NCH = BPW // C
NSLOT = 3
LA = 2
KCH = D // LANES


def _dot_chunk(ubuf, ibuf, xui_v, slot, c):
    last = lax.iota(jnp.int32, LANES) == (LANES - 1)

    def row_body(r):
        acc = ubuf[slot, r, pl.ds(0, LANES)] * ibuf[slot, r, pl.ds(0, LANES)]
        for k in range(1, KCH):
            acc = acc + (ubuf[slot, r, pl.ds(k * LANES, LANES)]
                         * ibuf[slot, r, pl.ds(k * LANES, LANES)])
        s = plsc.cumsum(acc)
        plsc.store_scatter(xui_v, [jnp.full((LANES,), c * C, jnp.int32) + r],
                           s, mask=last)

    plsc.parallel_loop(0, C, unroll=4)(row_body)


@functools.partial(
    pl.kernel,
    out_type=(
        jax.ShapeDtypeStruct((B,), jnp.float32),
        jax.ShapeDtypeStruct((B, D), jnp.float32),
        jax.ShapeDtypeStruct((B, D), jnp.float32),
    ),
    mesh=plsc.VectorSubcoreMesh(core_axis_name="c", subcore_axis_name="s"),
    compiler_params=pltpu.CompilerParams(needs_layout_passes=False),
    scratch_types=[
        pltpu.VMEM((BPW,), jnp.int32),
        pltpu.VMEM((BPW,), jnp.int32),
        pltpu.VMEM((NSLOT, C, D), jnp.float32),
        pltpu.VMEM((NSLOT, C, D), jnp.float32),
        pltpu.VMEM((BPW,), jnp.float32),
        [pltpu.SemaphoreType.DMA] * NSLOT,
        [pltpu.SemaphoreType.DMA] * NSLOT,
        [pltpu.SemaphoreType.DMA] * NSLOT,
        [pltpu.SemaphoreType.DMA] * NSLOT,
        pltpu.SemaphoreType.DMA,
    ],
)
def _ngcf_sc(user_h, item_h, Gu_h, Gi_h, xui_o, gu_o, gi_o,
             uidx, iidx, ubuf, ibuf, xui_v,
             gsems_u, gsems_i, wsems_u, wsems_i, isem):
    wid = lax.axis_index("s") * NC + lax.axis_index("c")
    base = wid * BPW

    cu = pltpu.async_copy(user_h.at[pl.ds(base, BPW)], uidx, isem)
    ci = pltpu.async_copy(item_h.at[pl.ds(base, BPW)], iidx, isem)
    cu.wait()
    ci.wait()

    def gather_u(c):
        slot = c % NSLOT
        return pltpu.async_copy(Gu_h.at[uidx.at[pl.ds(c * C, C)]],
                                ubuf.at[slot], gsems_u[slot])

    def gather_i(c):
        slot = c % NSLOT
        return pltpu.async_copy(Gi_h.at[iidx.at[pl.ds(c * C, C)]],
                                ibuf.at[slot], gsems_i[slot])

    def write_u(c):
        slot = c % NSLOT
        return pltpu.async_copy(ubuf.at[slot],
                                gu_o.at[pl.ds(base + c * C, C)], wsems_u[slot])

    def write_i(c):
        slot = c % NSLOT
        return pltpu.async_copy(ibuf.at[slot],
                                gi_o.at[pl.ds(base + c * C, C)], wsems_i[slot])

    gpend = {c: (gather_u(c), gather_i(c)) for c in range(LA)}
    wpend = {}
    for c in range(NCH):
        gu, gi = gpend.pop(c)
        gu.wait()
        wu = write_u(c)
        gi.wait()
        wi = write_i(c)
        wpend[c] = (wu, wi)
        if c + LA < NCH:
            prev = c + LA - NSLOT
            if prev in wpend:
                pwu, pwi = wpend.pop(prev)
                pwu.wait()
                pwi.wait()
            gpend[c + LA] = (gather_u(c + LA), gather_i(c + LA))
        _dot_chunk(ubuf, ibuf, xui_v, c % NSLOT, c)

    for wu, wi in wpend.values():
        wu.wait()
        wi.wait()
    pltpu.sync_copy(xui_v, xui_o.at[pl.ds(base, BPW)])


def kernel(user, item, Gu, Gi):
    xui, gamma_u, gamma_i = _ngcf_sc(user, item, Gu, Gi)
    return (xui, gamma_u, gamma_i)
